# baseline (speedup 1.0000x reference)
import functools
import math

import jax
import jax.numpy as jnp
from jax import lax
from jax.experimental import pallas as pl
from jax.experimental.pallas import tpu as pltpu
from jax.experimental.pallas import tpu_sc as plsc

B = 16384
NC, NS = 2, 16
NW = NC * NS
BPW = B // NW
CHUNK = 128
NCH = BPW // CHUNK
L = 16

N_ITEMS = 1000000
NWIN = N_ITEMS // 128
KMAX = NWIN // NW + 1

D_ITEM, D_CAT = 32, 16
H, OUT = 256, 64
_BN = 1.0 / math.sqrt(1.0 + 1e-5)

_TAB = ((2, 3, D_ITEM), (3, 7, D_CAT), (3, 7, D_CAT), (3, 7, D_CAT))

_sc_mesh = plsc.VectorSubcoreMesh(
    core_axis_name="c", subcore_axis_name="s", num_cores=NC, num_subcores=NS)


def _sc_pack_body(tT, tail, out, bufa, bufb, oa, ob, sa, sb):
    wid = lax.axis_index("s") * NC + lax.axis_index("c")
    iota = lax.iota(jnp.int32, L)
    r4 = lax.shift_right_logical(iota, 2)
    c4 = lax.bitwise_and(iota, 3) * D_ITEM

    def valid(k):
        return wid + NW * k < NWIN

    def win(k):
        return wid + NW * k

    def fire(k, buf, sem):
        @pl.when(valid(k))
        def _():
            off = pl.multiple_of(win(k) * 128, 128)
            pltpu.make_async_copy(tT.at[:, pl.ds(off, 128)], buf, sem).start()

    def process(k, buf, obuf, sem):
        @pl.when(valid(k))
        def _():
            pltpu.make_async_copy(tT.at[:, pl.ds(0, 128)], buf, sem).wait()
            for d in range(D_ITEM):
                for i in range(8):
                    v = buf[d, pl.ds(i * L, L)]
                    plsc.store_scatter(obuf, [4 * i + r4, c4 + d], v)
            pltpu.sync_copy(obuf, out.at[pl.ds(win(k) * 32, 32)])

    fire(0, bufa, sa)

    def body(k2, carry):
        fire(2 * k2 + 1, bufb, sb)
        process(2 * k2, bufa, oa, sa)
        fire(2 * k2 + 2, bufa, sa)
        process(2 * k2 + 1, bufb, ob, sb)
        return carry

    lax.fori_loop(0, (KMAX + 1) // 2, body, 0)

    @pl.when(wid == NW - 1)
    def _():
        pltpu.sync_copy(tail, out.at[pl.ds(NWIN * 32, 16)])


_sc_pack = pl.kernel(
    _sc_pack_body,
    out_type=[jax.ShapeDtypeStruct((N_ITEMS // 4, 128), jnp.float32)],
    mesh=_sc_mesh,
    scratch_types=(
        [pltpu.VMEM((D_ITEM, 128), jnp.float32) for _ in range(4)]
        + [pltpu.SemaphoreType.DMA for _ in range(2)]),
    compiler_params=pltpu.CompilerParams(needs_layout_passes=False),
)


def _sc_gather_body(c0, c1, c2, c3, c4, t0, t1, t2, t3, t4,
                    e0, e1, e2, e3, e4,
                    raw0, raw1, raw2, raw3, raw4,
                    si0, si1, si2, si3,
                    ba, bb, b4,
                    o32, o16, o16p,
                    sa, sb):
    wid = lax.axis_index("s") * NC + lax.axis_index("c")
    base = wid * BPW
    raws = (raw0, raw1, raw2, raw3, raw4)
    sidx = (si0, si1, si2, si3)
    bufs = (ba, bb)
    ehbm = (e0, e1, e2, e3)
    sems = (sa, sb)

    for cref, rref in zip((c0, c1, c2, c3, c4), raws):
        pltpu.sync_copy(cref.at[pl.ds(base, BPW)], rref)
    pltpu.sync_copy(t4, b4)

    for t in range(4):
        sh = _TAB[t][0]
        for j in range(NCH):
            for k in range(CHUNK // L):
                v = raws[t][pl.ds(j * CHUNK + k * L, L)]
                sidx[t][j, pl.ds(k * L, L)] = lax.shift_right_logical(
                    v, jnp.int32(sh))

    def extract(t, j, buf, out):
        _, msk, width = _TAB[t]

        def grp(g, carry):
            rows = lax.iota(jnp.int32, L) + g * L
            rv = raws[t][pl.ds(j * CHUNK + g * L, L)]
            colbase = lax.bitwise_and(rv, jnp.int32(msk)) * width
            for jj in range(width):
                x = plsc.load_gather(buf, [rows, colbase + jj])
                plsc.store_scatter(out, [rows, jnp.full((L,), jj, jnp.int32)],
                                   x)
            return carry

        lax.fori_loop(0, CHUNK // L, grp, 0)

    def extract_price(j, out):
        def grp(g, carry):
            rows = lax.iota(jnp.int32, L) + g * L
            rv = raws[4][pl.ds(j * CHUNK + g * L, L)]
            colbase = rv * D_CAT
            zero = jnp.zeros((L,), jnp.int32)
            for jj in range(D_CAT):
                x = plsc.load_gather(b4, [zero, colbase + jj])
                plsc.store_scatter(out, [rows, jnp.full((L,), jj, jnp.int32)],
                                   x)
            return carry

        lax.fori_loop(0, CHUNK // L, grp, 0)

    tabs = (t0, t1, t2, t3)
    steps = [(j, t) for j in range(NCH) for t in range(4)]
    h = [None, None]

    def fire(s):
        j, t = steps[s]
        h[s % 2] = pltpu.async_copy(tabs[t].at[sidx[t].at[j]], bufs[s % 2],
                                    sems[s % 2])

    def drain(s):
        j, t = steps[s]
        h[s % 2].wait()
        out = o32 if t == 0 else o16
        extract(t, j, bufs[s % 2], out)
        pltpu.sync_copy(out, ehbm[t].at[pl.ds(base + j * CHUNK, CHUNK)])

    fire(0)
    for j in range(NCH):
        extract_price(j, o16p)
        pltpu.sync_copy(o16p, e4.at[pl.ds(base + j * CHUNK, CHUNK)])
    for s in range(1, len(steps)):
        fire(s)
        drain(s - 1)
    drain(len(steps) - 1)


_sc_gather = pl.kernel(
    _sc_gather_body,
    out_type=[jax.ShapeDtypeStruct((B, D_ITEM), jnp.float32)]
    + [jax.ShapeDtypeStruct((B, D_CAT), jnp.float32) for _ in range(4)],
    mesh=_sc_mesh,
    scratch_types=(
        [pltpu.VMEM((BPW,), jnp.int32) for _ in range(5)]
        + [pltpu.VMEM((NCH, CHUNK), jnp.int32) for _ in range(4)]
        + [pltpu.VMEM((CHUNK, 128), jnp.float32) for _ in range(2)]
        + [pltpu.VMEM((1, 128), jnp.float32)]
        + [pltpu.VMEM((CHUNK, D_ITEM), jnp.float32)]
        + [pltpu.VMEM((CHUNK, D_CAT), jnp.float32) for _ in range(2)]
        + [pltpu.SemaphoreType.DMA for _ in range(2)]),
    compiler_params=pltpu.CompilerParams(needs_layout_passes=False),
)


def _mlp_body(e0, e1, e2, e3, e4, dnT, w1a, w1b, w1c, w1d, w1e, w1f,
              b1, gm, bt, w2t, b2, out):
    h = jnp.dot(e0[...], w1a[...], preferred_element_type=jnp.float32)
    h = h + jnp.dot(e1[...], w1b[...], preferred_element_type=jnp.float32)
    h = h + jnp.dot(e2[...], w1c[...], preferred_element_type=jnp.float32)
    h = h + jnp.dot(e3[...], w1d[...], preferred_element_type=jnp.float32)
    h = h + jnp.dot(e4[...], w1e[...], preferred_element_type=jnp.float32)
    h = h + lax.dot_general(dnT[...], w1f[...], (((0,), (0,)), ((), ())),
                            preferred_element_type=jnp.float32)
    h = (h + b1[...]) * (_BN * gm[...]) + bt[...]
    h = jnp.maximum(h, 0.0)
    o = lax.dot_general(h, w2t[...], (((1,), (1,)), ((), ())),
                        preferred_element_type=jnp.float32) + b2[...]
    nrm = jnp.sqrt(jnp.sum(o * o, axis=1, keepdims=True))
    out[...] = o / jnp.maximum(nrm, 1e-12)


def _mlp(e0, e1, e2, e3, e4, dnT, w1a, w1b, w1c, w1d, w1e, w1f,
         b1, gm, bt, w2t, b2, block_rows=2048):
    grid = (B // block_rows,)

    def row_spec(d):
        return pl.BlockSpec((block_rows, d), lambda i: (i, 0))

    def full_spec(shape):
        return pl.BlockSpec(shape, lambda i: (0,) * len(shape))

    return pl.pallas_call(
        _mlp_body,
        grid=grid,
        in_specs=[
            row_spec(D_ITEM), row_spec(D_CAT), row_spec(D_CAT),
            row_spec(D_CAT), row_spec(D_CAT),
            pl.BlockSpec((3, block_rows), lambda i: (0, i)),
            full_spec((D_ITEM, H)), full_spec((D_CAT, H)),
            full_spec((D_CAT, H)), full_spec((D_CAT, H)),
            full_spec((D_CAT, H)), full_spec((3, H)),
            full_spec((1, H)), full_spec((1, H)), full_spec((1, H)),
            full_spec((OUT, H)), full_spec((1, OUT)),
        ],
        out_specs=pl.BlockSpec((block_rows, OUT), lambda i: (i, 0)),
        out_shape=jax.ShapeDtypeStruct((B, OUT), jnp.float32),
    )(e0, e1, e2, e3, e4, dnT, w1a, w1b, w1c, w1d, w1e, w1f,
      b1, gm, bt, w2t, b2)


def kernel(item_cat, item_dense, item_emb, cat_l1_emb, cat_l2_emb,
           brand_emb, price_emb, W1, b1, gamma, beta, W2, b2):
    ic = item_cat.astype(jnp.int32)
    c0, c1, c2, c3, c4 = (ic[:, j] for j in range(5))

    tail = item_emb[NWIN * 128:].reshape(16, 128)
    (item128,) = _sc_pack(item_emb.T, tail)

    l1_128 = cat_l1_emb.reshape(-1, 128)
    l2_128 = cat_l2_emb.reshape(-1, 128)
    brand128 = brand_emb.reshape(-1, 128)
    price16 = jnp.pad(price_emb, ((0, 0), (0, 8))).reshape(1, 128)

    e0, e1, e2, e3, e4 = _sc_gather(
        c0, c1, c2, c3, c4, item128, l1_128, l2_128, brand128, price16)

    w1a = W1[0:32]
    w1b = W1[32:48]
    w1c = W1[48:64]
    w1d = W1[64:80]
    w1e = jnp.pad(W1[80:88], ((0, 8), (0, 0)))
    w1f = W1[88:91]

    return _mlp(e0, e1, e2, e3, e4, item_dense.T,
                w1a, w1b, w1c, w1d, w1e, w1f,
                b1.reshape(1, H), gamma.reshape(1, H), beta.reshape(1, H),
                W2.T, b2.reshape(1, OUT))


# speedup vs baseline: 1.2775x; 1.2775x over previous
import functools
import math

import jax
import jax.numpy as jnp
from jax import lax
from jax.experimental import pallas as pl
from jax.experimental.pallas import tpu as pltpu
from jax.experimental.pallas import tpu_sc as plsc

B = 16384
NC, NS = 2, 16
NW = NC * NS
BPW = B // NW
CHUNK = 128
NCH = BPW // CHUNK
L = 16

N_ITEMS = 1000000
NWIN = N_ITEMS // 128
KMAX = NWIN // NW + 1

D_ITEM, D_CAT = 32, 16
H, OUT = 256, 64
_BN = 1.0 / math.sqrt(1.0 + 1e-5)

_TAB = ((2, 3, D_ITEM), (3, 7, D_CAT), (3, 7, D_CAT), (3, 7, D_CAT))

_sc_mesh = plsc.VectorSubcoreMesh(
    core_axis_name="c", subcore_axis_name="s", num_cores=NC, num_subcores=NS)


def _sc_pack_body(tT, tail, out, bufa, bufb, oa, ob, sa, sb):
    wid = lax.axis_index("s") * NC + lax.axis_index("c")
    iota = lax.iota(jnp.int32, L)
    r4 = lax.shift_right_logical(iota, 2)
    c4 = lax.bitwise_and(iota, 3) * D_ITEM

    def valid(k):
        return wid + NW * k < NWIN

    def win(k):
        return wid + NW * k

    def fire(k, buf, sem):
        @pl.when(valid(k))
        def _():
            off = pl.multiple_of(win(k) * 128, 128)
            pltpu.make_async_copy(tT.at[:, pl.ds(off, 128)], buf, sem).start()

    def process(k, buf, obuf, sem):
        @pl.when(valid(k))
        def _():
            pltpu.make_async_copy(tT.at[:, pl.ds(0, 128)], buf, sem).wait()
            for d in range(D_ITEM):
                for i in range(8):
                    v = buf[d, pl.ds(i * L, L)]
                    plsc.store_scatter(obuf, [4 * i + r4, c4 + d], v)
            pltpu.sync_copy(obuf, out.at[pl.ds(win(k) * 32, 32)])

    fire(0, bufa, sa)

    def body(k2, carry):
        fire(2 * k2 + 1, bufb, sb)
        process(2 * k2, bufa, oa, sa)
        fire(2 * k2 + 2, bufa, sa)
        process(2 * k2 + 1, bufb, ob, sb)
        return carry

    lax.fori_loop(0, (KMAX + 1) // 2, body, 0)

    @pl.when(wid == NW - 1)
    def _():
        pltpu.sync_copy(tail, out.at[pl.ds(NWIN * 32, 16)])


_sc_pack = pl.kernel(
    _sc_pack_body,
    out_type=[jax.ShapeDtypeStruct((N_ITEMS // 4, 128), jnp.float32)],
    mesh=_sc_mesh,
    scratch_types=(
        [pltpu.VMEM((D_ITEM, 128), jnp.float32) for _ in range(4)]
        + [pltpu.SemaphoreType.DMA for _ in range(2)]),
    compiler_params=pltpu.CompilerParams(needs_layout_passes=False),
)


def _sc_gather_body(c0, c1, c2, c3, c4, t0, t1, t2, t3, t4,
                    e0, e1, e2, e3, e4,
                    raw0, raw1, raw2, raw3, raw4,
                    si0, si1, si2, si3,
                    ba, bb, b4,
                    o32, o16, o16p,
                    sa, sb):
    wid = lax.axis_index("s") * NC + lax.axis_index("c")
    base = wid * BPW
    raws = (raw0, raw1, raw2, raw3, raw4)
    sidx = (si0, si1, si2, si3)
    bufs = (ba, bb)
    ehbm = (e0, e1, e2, e3)
    sems = (sa, sb)

    for cref, rref in zip((c0, c1, c2, c3, c4), raws):
        pltpu.sync_copy(cref.at[pl.ds(base, BPW)], rref)
    pltpu.sync_copy(t4, b4)

    for t in range(4):
        sh = _TAB[t][0]
        for j in range(NCH):
            for k in range(CHUNK // L):
                v = raws[t][pl.ds(j * CHUNK + k * L, L)]
                sidx[t][j, pl.ds(k * L, L)] = lax.shift_right_logical(
                    v, jnp.int32(sh))

    def extract(t, j, buf, out):
        _, msk, width = _TAB[t]

        def grp(g, carry):
            rows = lax.iota(jnp.int32, L) + g * L
            rv = raws[t][pl.ds(j * CHUNK + g * L, L)]
            colbase = lax.bitwise_and(rv, jnp.int32(msk)) * width
            for jj in range(width):
                x = plsc.load_gather(buf, [rows, colbase + jj])
                plsc.store_scatter(out, [rows, jnp.full((L,), jj, jnp.int32)],
                                   x)
            return carry

        lax.fori_loop(0, CHUNK // L, grp, 0)

    def extract_price(j, out):
        def grp(g, carry):
            rows = lax.iota(jnp.int32, L) + g * L
            rv = raws[4][pl.ds(j * CHUNK + g * L, L)]
            colbase = rv * D_CAT
            zero = jnp.zeros((L,), jnp.int32)
            for jj in range(D_CAT):
                x = plsc.load_gather(b4, [zero, colbase + jj])
                plsc.store_scatter(out, [rows, jnp.full((L,), jj, jnp.int32)],
                                   x)
            return carry

        lax.fori_loop(0, CHUNK // L, grp, 0)

    tabs = (t0, t1, t2, t3)
    steps = [(j, t) for j in range(NCH) for t in range(4)]
    h = [None, None]

    def fire(s):
        j, t = steps[s]
        h[s % 2] = pltpu.async_copy(tabs[t].at[sidx[t].at[j]], bufs[s % 2],
                                    sems[s % 2])

    def drain(s):
        j, t = steps[s]
        h[s % 2].wait()
        out = o32 if t == 0 else o16
        extract(t, j, bufs[s % 2], out)
        pltpu.sync_copy(out, ehbm[t].at[pl.ds(base + j * CHUNK, CHUNK)])

    fire(0)
    for j in range(NCH):
        extract_price(j, o16p)
        pltpu.sync_copy(o16p, e4.at[pl.ds(base + j * CHUNK, CHUNK)])
    for s in range(1, len(steps)):
        fire(s)
        drain(s - 1)
    drain(len(steps) - 1)


_sc_gather = pl.kernel(
    _sc_gather_body,
    out_type=[jax.ShapeDtypeStruct((B, D_ITEM), jnp.float32)]
    + [jax.ShapeDtypeStruct((B, D_CAT), jnp.float32) for _ in range(4)],
    mesh=_sc_mesh,
    scratch_types=(
        [pltpu.VMEM((BPW,), jnp.int32) for _ in range(5)]
        + [pltpu.VMEM((NCH, CHUNK), jnp.int32) for _ in range(4)]
        + [pltpu.VMEM((CHUNK, 128), jnp.float32) for _ in range(2)]
        + [pltpu.VMEM((1, 128), jnp.float32)]
        + [pltpu.VMEM((CHUNK, D_ITEM), jnp.float32)]
        + [pltpu.VMEM((CHUNK, D_CAT), jnp.float32) for _ in range(2)]
        + [pltpu.SemaphoreType.DMA for _ in range(2)]),
    compiler_params=pltpu.CompilerParams(needs_layout_passes=False),
)


def _mlp_body(e0, e1, e2, e3, e4, dnT, w1a, w1b, w1c, w1d, w1e, w1f,
              b1, gm, bt, w2t, b2, out):
    h = jnp.dot(e0[...], w1a[...], preferred_element_type=jnp.float32)
    h = h + jnp.dot(e1[...], w1b[...], preferred_element_type=jnp.float32)
    h = h + jnp.dot(e2[...], w1c[...], preferred_element_type=jnp.float32)
    h = h + jnp.dot(e3[...], w1d[...], preferred_element_type=jnp.float32)
    h = h + jnp.dot(e4[...], w1e[...], preferred_element_type=jnp.float32)
    h = h + lax.dot_general(dnT[...], w1f[...], (((0,), (0,)), ((), ())),
                            preferred_element_type=jnp.float32)
    h = (h + b1[...]) * (_BN * gm[...]) + bt[...]
    h = jnp.maximum(h, 0.0)
    o = lax.dot_general(h, w2t[...], (((1,), (1,)), ((), ())),
                        preferred_element_type=jnp.float32) + b2[...]
    nrm = jnp.sqrt(jnp.sum(o * o, axis=1, keepdims=True))
    out[...] = o / jnp.maximum(nrm, 1e-12)


def _mlp(e0, e1, e2, e3, e4, dnT, w1a, w1b, w1c, w1d, w1e, w1f,
         b1, gm, bt, w2t, b2, block_rows=2048):
    grid = (B // block_rows,)

    def row_spec(d):
        return pl.BlockSpec((block_rows, d), lambda i: (i, 0))

    def full_spec(shape):
        return pl.BlockSpec(shape, lambda i: (0,) * len(shape))

    return pl.pallas_call(
        _mlp_body,
        grid=grid,
        in_specs=[
            row_spec(D_ITEM), row_spec(D_CAT), row_spec(D_CAT),
            row_spec(D_CAT), row_spec(D_CAT),
            pl.BlockSpec((3, block_rows), lambda i: (0, i)),
            full_spec((D_ITEM, H)), full_spec((D_CAT, H)),
            full_spec((D_CAT, H)), full_spec((D_CAT, H)),
            full_spec((D_CAT, H)), full_spec((3, H)),
            full_spec((1, H)), full_spec((1, H)), full_spec((1, H)),
            full_spec((OUT, H)), full_spec((1, OUT)),
        ],
        out_specs=pl.BlockSpec((block_rows, OUT), lambda i: (i, 0)),
        out_shape=jax.ShapeDtypeStruct((B, OUT), jnp.float32),
    )(e0, e1, e2, e3, e4, dnT, w1a, w1b, w1c, w1d, w1e, w1f,
      b1, gm, bt, w2t, b2)


def kernel(item_cat, item_dense, item_emb, cat_l1_emb, cat_l2_emb,
           brand_emb, price_emb, W1, b1, gamma, beta, W2, b2):
    ic = item_cat.astype(jnp.int32)
    c0, c1, c2, c3, c4 = (ic[:, j] for j in range(5))

    item128 = item_emb.reshape(N_ITEMS // 4, 128)

    l1_128 = cat_l1_emb.reshape(-1, 128)
    l2_128 = cat_l2_emb.reshape(-1, 128)
    brand128 = brand_emb.reshape(-1, 128)
    price16 = jnp.pad(price_emb, ((0, 0), (0, 8))).reshape(1, 128)

    e0, e1, e2, e3, e4 = _sc_gather(
        c0, c1, c2, c3, c4, item128, l1_128, l2_128, brand128, price16)

    w1a = W1[0:32]
    w1b = W1[32:48]
    w1c = W1[48:64]
    w1d = W1[64:80]
    w1e = jnp.pad(W1[80:88], ((0, 8), (0, 0)))
    w1f = W1[88:91]

    return _mlp(e0, e1, e2, e3, e4, item_dense.T,
                w1a, w1b, w1c, w1d, w1e, w1f,
                b1.reshape(1, H), gamma.reshape(1, H), beta.reshape(1, H),
                W2.T, b2.reshape(1, OUT))
